# Initial kernel scaffold; baseline (speedup 1.0000x reference)
import jax
import jax.numpy as jnp
from jax.experimental import pallas as pl


def kernel(X, edge_index, edge_weight, W, b):
    raise NotImplementedError("write your pallas kernel here")


# speedup vs baseline: 3.4328x; 3.4328x over previous
import functools

import jax
import jax.numpy as jnp
from jax import lax
from jax.experimental import pallas as pl
from jax.experimental.pallas import tpu as pltpu
from jax.experimental.pallas import tpu_sc as plsc

N_NODES = 10000
D = 128
DH = D // 2
NC = 2
NS = 16
CHUNK = 128
N_CHUNKS = 160
E_PAD = NS * N_CHUNKS * CHUNK
N_ACC = 10240
ROWS_PER_TILE = N_ACC // NS
ZROWS = 128


def _sc_scatter(T, src, dst, ew):
    mesh = plsc.VectorSubcoreMesh(
        core_axis_name="c", subcore_axis_name="s",
        num_cores=NC, num_subcores=NS)

    @functools.partial(
        pl.kernel,
        out_type=jax.ShapeDtypeStruct((NC, N_ACC, DH), jnp.float32),
        mesh=mesh,
        scratch_types=[
            pltpu.VMEM((N_CHUNKS, CHUNK), jnp.int32),
            pltpu.VMEM((N_CHUNKS, CHUNK), jnp.int32),
            pltpu.VMEM((N_CHUNKS, CHUNK), jnp.float32),
            pltpu.VMEM((CHUNK, DH), jnp.float32),
            pltpu.VMEM((ZROWS, DH), jnp.float32),
            pltpu.VMEM_SHARED((N_ACC, DH), jnp.float32),
            pltpu.SemaphoreType.DMA,
        ],
        compiler_params=pltpu.CompilerParams(use_tc_tiling_on_sc=False),
    )
    def k(t_hbm, src_hbm, dst_hbm, ew_hbm, out_hbm,
          src_v, dst_v, ew_v, rows_v, zero_v, acc, sem):
        c = lax.axis_index("c")
        s = lax.axis_index("s")

        def zrow(i, carry):
            for v in range(DH // 16):
                zero_v[i, pl.ds(16 * v, 16)] = jnp.zeros((16,), jnp.float32)
            return carry
        lax.fori_loop(0, ZROWS, zrow, 0)
        base = s * ROWS_PER_TILE
        for t in range(ROWS_PER_TILE // ZROWS):
            pltpu.sync_copy(zero_v, acc.at[pl.ds(base + t * ZROWS, ZROWS)])

        pltpu.sync_copy(src_hbm.at[s], src_v)
        pltpu.sync_copy(dst_hbm.at[s], dst_v)
        pltpu.sync_copy(ew_hbm.at[s], ew_v)
        row0 = c * N_NODES

        def rebase(i, carry):
            for v in range(CHUNK // 16):
                sl = pl.ds(16 * v, 16)
                src_v[i, sl] = src_v[i, sl] + row0
            return carry
        lax.fori_loop(0, N_CHUNKS, rebase, 0)
        plsc.subcore_barrier()

        def chunk(j, carry):
            pltpu.async_copy(t_hbm.at[src_v.at[j]], rows_v, sem).wait()

            def group(g, gcarry):
                wv = ew_v[j, pl.ds(g * 16, 16)]
                for i in range(16):
                    e = g * 16 + i
                    w = wv[i]
                    for v in range(DH // 16):
                        sl = pl.ds(16 * v, 16)
                        rows_v[e, sl] = rows_v[e, sl] * w
                return gcarry
            lax.fori_loop(0, CHUNK // 16, group, 0)

            pltpu.sync_copy(rows_v, acc.at[dst_v.at[j]], add=True)
            return carry
        lax.fori_loop(0, N_CHUNKS, chunk, 0)

        plsc.subcore_barrier()
        for t in range(ROWS_PER_TILE // ZROWS):
            lo = base + t * ZROWS
            pltpu.sync_copy(acc.at[pl.ds(lo, ZROWS)],
                            out_hbm.at[c, pl.ds(lo, ZROWS)])

    return k(T, src, dst, ew)


def _tc_body(p0_ref, p1_ref, w0_ref, w1_ref, b_ref, o_ref):
    o_ref[...] = (
        lax.dot_general(p0_ref[...], w0_ref[...], (((1,), (1,)), ((), ())),
                        preferred_element_type=jnp.float32)
        + lax.dot_general(p1_ref[...], w1_ref[...], (((1,), (1,)), ((), ())),
                          preferred_element_type=jnp.float32)
        + b_ref[...])


def _tc_linear(p0, p1, w0, w1, b2d):
    rows = 1000
    return pl.pallas_call(
        _tc_body,
        grid=(N_NODES // rows,),
        in_specs=[
            pl.BlockSpec((rows, DH), lambda i: (i, 0)),
            pl.BlockSpec((rows, DH), lambda i: (i, 0)),
            pl.BlockSpec((D, DH), lambda i: (0, 0)),
            pl.BlockSpec((D, DH), lambda i: (0, 0)),
            pl.BlockSpec((1, D), lambda i: (0, 0)),
        ],
        out_specs=pl.BlockSpec((rows, D), lambda i: (i, 0)),
        out_shape=jax.ShapeDtypeStruct((N_NODES, D), jnp.float32),
    )(p0, p1, w0, w1, b2d)


def kernel(X, edge_index, edge_weight, W, b):
    src = edge_index[1].astype(jnp.int32)
    dst = edge_index[0].astype(jnp.int32)
    ew = edge_weight.astype(jnp.float32)
    pad = E_PAD - src.shape[0]
    src = jnp.pad(src, (0, pad)).reshape(NS, N_CHUNKS, CHUNK)
    dst = jnp.pad(dst, (0, pad)).reshape(NS, N_CHUNKS, CHUNK)
    ew = jnp.pad(ew, (0, pad)).reshape(NS, N_CHUNKS, CHUNK)
    T = jnp.concatenate([X[:, :DH], X[:, DH:]], axis=0)
    part = _sc_scatter(T, src, dst, ew)
    return _tc_linear(part[0, :N_NODES], part[1, :N_NODES],
                      W[:, :DH], W[:, DH:], b.reshape(1, D))


# speedup vs baseline: 4.8834x; 1.4226x over previous
import functools

import jax
import jax.numpy as jnp
from jax import lax
from jax.experimental import pallas as pl
from jax.experimental.pallas import tpu as pltpu
from jax.experimental.pallas import tpu_sc as plsc

N_NODES = 10000
D = 128
DH = D // 2
NC = 2
NS = 16
CHUNK = 128
N_CHUNKS = 160
E_PAD = NS * N_CHUNKS * CHUNK
N_ACC = 10240
ROWS_PER_TILE = N_ACC // NS
ZROWS = 128


def _sc_scatter(T, src, dst, ew):
    mesh = plsc.VectorSubcoreMesh(
        core_axis_name="c", subcore_axis_name="s",
        num_cores=NC, num_subcores=NS)

    @functools.partial(
        pl.kernel,
        out_type=jax.ShapeDtypeStruct((NC, N_ACC, DH), jnp.float32),
        mesh=mesh,
        scratch_types=[
            pltpu.VMEM((N_CHUNKS, CHUNK), jnp.int32),
            pltpu.VMEM((N_CHUNKS, CHUNK), jnp.int32),
            pltpu.VMEM((N_CHUNKS, CHUNK), jnp.float32),
            pltpu.VMEM((CHUNK, DH), jnp.float32),
            pltpu.VMEM((CHUNK, DH), jnp.float32),
            pltpu.VMEM((ZROWS, DH), jnp.float32),
            pltpu.VMEM_SHARED((N_ACC, DH), jnp.float32),
            pltpu.SemaphoreType.DMA,
            pltpu.SemaphoreType.DMA,
            pltpu.SemaphoreType.DMA,
            pltpu.SemaphoreType.DMA,
        ],
        compiler_params=pltpu.CompilerParams(use_tc_tiling_on_sc=False),
    )
    def k(t_hbm, src_hbm, dst_hbm, ew_hbm, out_hbm,
          src_v, dst_v, ew_v, rows0_v, rows1_v, zero_v, acc,
          sem_g0, sem_g1, sem_s0, sem_s1):
        bufs = (rows0_v, rows1_v)
        sems_g = (sem_g0, sem_g1)
        sems_s = (sem_s0, sem_s1)
        c = lax.axis_index("c")
        s = lax.axis_index("s")

        def zrow(i, carry):
            for v in range(DH // 16):
                zero_v[i, pl.ds(16 * v, 16)] = jnp.zeros((16,), jnp.float32)
            return carry
        lax.fori_loop(0, ZROWS, zrow, 0)
        base = s * ROWS_PER_TILE
        for t in range(ROWS_PER_TILE // ZROWS):
            pltpu.sync_copy(zero_v, acc.at[pl.ds(base + t * ZROWS, ZROWS)])

        pltpu.sync_copy(src_hbm.at[s], src_v)
        pltpu.sync_copy(dst_hbm.at[s], dst_v)
        pltpu.sync_copy(ew_hbm.at[s], ew_v)
        row0 = c * N_NODES

        def rebase(i, carry):
            for v in range(CHUNK // 16):
                sl = pl.ds(16 * v, 16)
                src_v[i, sl] = src_v[i, sl] + row0
            return carry
        lax.fori_loop(0, N_CHUNKS, rebase, 0)
        plsc.subcore_barrier()

        def scale(j, rows_v):
            def group(g, gcarry):
                wv = ew_v[j, pl.ds(g * 16, 16)]
                for i in range(16):
                    e = g * 16 + i
                    w = wv[i]
                    for v in range(DH // 16):
                        sl = pl.ds(16 * v, 16)
                        rows_v[e, sl] = rows_v[e, sl] * w
                return gcarry
            lax.fori_loop(0, CHUNK // 16, group, 0)

        pltpu.async_copy(t_hbm.at[src_v.at[0]], bufs[0], sems_g[0])

        def pair(g, carry):
            for b in range(2):
                j = 2 * g + b
                nb = 1 - b
                jn = jnp.minimum(j + 1, N_CHUNKS - 1)

                @pl.when(j >= 1)
                def _():
                    pltpu.make_async_copy(
                        bufs[nb], acc.at[dst_v.at[j]], sems_s[nb]).wait()
                pltpu.async_copy(t_hbm.at[src_v.at[jn]], bufs[nb], sems_g[nb])

                pltpu.make_async_copy(
                    t_hbm.at[src_v.at[j]], bufs[b], sems_g[b]).wait()
                scale(j, bufs[b])
                pltpu.async_copy(bufs[b], acc.at[dst_v.at[j]], sems_s[b],
                                 add=True)
            return carry
        lax.fori_loop(0, N_CHUNKS // 2, pair, 0)
        pltpu.make_async_copy(
            t_hbm.at[src_v.at[0]], bufs[0], sems_g[0]).wait()
        pltpu.make_async_copy(bufs[1], acc.at[dst_v.at[0]], sems_s[1]).wait()

        plsc.subcore_barrier()
        for t in range(ROWS_PER_TILE // ZROWS):
            lo = base + t * ZROWS
            pltpu.sync_copy(acc.at[pl.ds(lo, ZROWS)],
                            out_hbm.at[c, pl.ds(lo, ZROWS)])

    return k(T, src, dst, ew)


def _tc_body(p0_ref, p1_ref, w0_ref, w1_ref, b_ref, o_ref):
    o_ref[...] = (
        lax.dot_general(p0_ref[...], w0_ref[...], (((1,), (1,)), ((), ())),
                        preferred_element_type=jnp.float32)
        + lax.dot_general(p1_ref[...], w1_ref[...], (((1,), (1,)), ((), ())),
                          preferred_element_type=jnp.float32)
        + b_ref[...])


def _tc_linear(p0, p1, w0, w1, b2d):
    rows = 1000
    return pl.pallas_call(
        _tc_body,
        grid=(N_NODES // rows,),
        in_specs=[
            pl.BlockSpec((rows, DH), lambda i: (i, 0)),
            pl.BlockSpec((rows, DH), lambda i: (i, 0)),
            pl.BlockSpec((D, DH), lambda i: (0, 0)),
            pl.BlockSpec((D, DH), lambda i: (0, 0)),
            pl.BlockSpec((1, D), lambda i: (0, 0)),
        ],
        out_specs=pl.BlockSpec((rows, D), lambda i: (i, 0)),
        out_shape=jax.ShapeDtypeStruct((N_NODES, D), jnp.float32),
    )(p0, p1, w0, w1, b2d)


def kernel(X, edge_index, edge_weight, W, b):
    src = edge_index[1].astype(jnp.int32)
    dst = edge_index[0].astype(jnp.int32)
    ew = edge_weight.astype(jnp.float32)
    pad = E_PAD - src.shape[0]
    src = jnp.pad(src, (0, pad)).reshape(NS, N_CHUNKS, CHUNK)
    dst = jnp.pad(dst, (0, pad)).reshape(NS, N_CHUNKS, CHUNK)
    ew = jnp.pad(ew, (0, pad)).reshape(NS, N_CHUNKS, CHUNK)
    T = jnp.concatenate([X[:, :DH], X[:, DH:]], axis=0)
    part = _sc_scatter(T, src, dst, ew)
    return _tc_linear(part[0, :N_NODES], part[1, :N_NODES],
                      W[:, :DH], W[:, DH:], b.reshape(1, D))
